# baseline (device time: 78428 ns/iter reference)
import numpy as np

import jax
import jax.numpy as jnp
from jax import lax
from jax.experimental import pallas as pl
from jax.experimental.pallas import tpu as pltpu

N_DEV = 8
SQ = 1024
SKV = 1024
H_PER = 8
DH = 128
D_MODEL = 1024
CHUNK = SQ // N_DEV
SCALE = 0.08838834764831843

KLISTS = [list(range(8)), [0, 1, 2]] + \
    [[0, c - 1, c, c + 1] for c in range(2, 7)] + [[0, 6, 7]]

def _block_mask(c):
    parts = []
    for kb in KLISTS[c]:
        qi = lax.broadcasted_iota(jnp.int32, (CHUNK, CHUNK), 0) + c * CHUNK
        ki = lax.broadcasted_iota(jnp.int32, (CHUNK, CHUNK), 1) + kb * CHUNK
        parts.append((jnp.abs(qi - ki) <= 128) | (ki < 32) | (qi < 32))
    return jnp.concatenate(parts, axis=1)

P1_OWNER = [4 * (c & 1) + 2 * ((c >> 2) & 1) + ((c >> 1) & 1)
            for c in range(N_DEV)]


def kernel(x, Wq, K_ext, V_ext, Wo):
    my = lax.axis_index("i")
    wq_my = lax.dynamic_slice(Wq, (0, my * H_PER * DH), (D_MODEL, H_PER * DH))
    wo_my = lax.dynamic_slice(Wo, (my * H_PER * DH, 0), (H_PER * DH, D_MODEL))
    x2 = x.reshape(SQ, D_MODEL)
    k3 = K_ext.reshape(SKV, H_PER, DH)
    v3 = V_ext.reshape(SKV, H_PER, DH)

    def body(x_ref, wq_ref, k_ref, v_ref, wo_ref, out_ref,
             qt_ref, kt_ref, vt_ref, stage_ref, a2a0_ref, a2a1_ref,
             gath_ref,
             a2a_send0, a2a_recv0, a2a_send1, a2a_recv1,
             ag_send_sems, ag_recv_sems):
        my_pos = lax.axis_index("i")
        b0 = my_pos & 1
        b1 = (my_pos >> 1) & 1
        b2 = (my_pos >> 2) & 1
        j1 = 4 * b1 + 2 * b0 + b2

        xb = x_ref[:, :].astype(jnp.bfloat16)
        for h in range(H_PER):
            qh = jax.lax.dot(
                xb, wq_ref[:, h * DH:(h + 1) * DH].astype(jnp.bfloat16),
                preferred_element_type=jnp.float32)
            qt_ref[h] = qh.astype(jnp.bfloat16)
            kt_ref[h] = k_ref[:, h, :].astype(jnp.bfloat16)
            vt_ref[h] = v_ref[:, h, :].astype(jnp.bfloat16)

        wo3 = wo_ref[:, :].reshape(H_PER, DH, D_MODEL)

        a2a_descs = []
        for c in range(N_DEV):
            kl = KLISTS[c]
            nk = len(kl) * CHUNK
            q_blk = qt_ref[:, c * CHUNK:(c + 1) * CHUNK, :]
            k_sub = jnp.concatenate(
                [kt_ref[:, kb * CHUNK:(kb + 1) * CHUNK, :] for kb in kl],
                axis=1)
            v_sub = jnp.concatenate(
                [vt_ref[:, kb * CHUNK:(kb + 1) * CHUNK, :] for kb in kl],
                axis=1)
            scores = lax.dot_general(
                q_blk, k_sub,
                dimension_numbers=(((2,), (2,)), ((0,), (0,))),
                preferred_element_type=jnp.float32,
            ) * SCALE
            mask = _block_mask(c)[None, :, :]
            scores = jnp.where(mask, scores, -1e9)
            m = jnp.max(scores, axis=2, keepdims=True)
            w = jnp.exp(scores - m)
            s = jnp.sum(w, axis=2, keepdims=True)
            w = (w / s).astype(jnp.bfloat16)
            ctx = lax.dot_general(
                w, v_sub,
                dimension_numbers=(((2,), (1,)), ((0,), (0,))),
                preferred_element_type=jnp.float32,
            ).astype(jnp.bfloat16)
            partial = lax.dot(
                ctx[0], wo3[0].astype(jnp.bfloat16),
                preferred_element_type=jnp.float32)
            for h in range(1, H_PER):
                partial = partial + lax.dot(
                    ctx[h], wo3[h].astype(jnp.bfloat16),
                    preferred_element_type=jnp.float32)
            stage_ref[c] = partial.astype(jnp.bfloat16)

            for p, dest in ((0, c), (1, P1_OWNER[c])):
                cs = p * 512
                recv_buf = a2a0_ref if p == 0 else a2a1_ref
                ssem = a2a_send0 if p == 0 else a2a_send1
                rsem = a2a_recv0 if p == 0 else a2a_recv1
                rdma = pltpu.make_async_remote_copy(
                    src_ref=stage_ref.at[c, :, pl.ds(cs, 512)],
                    dst_ref=recv_buf.at[my_pos],
                    send_sem=ssem.at[c],
                    recv_sem=rsem.at[my_pos],
                    device_id=(dest,),
                    device_id_type=pl.DeviceIdType.MESH,
                )
                is_self = my_pos == dest

                @pl.when(jnp.logical_not(is_self))
                def _():
                    rdma.start()

                @pl.when(is_self)
                def _():
                    recv_buf[my_pos] = stage_ref[c, :, pl.ds(cs, 512)]
                a2a_descs.append((rdma, is_self))

        for p in range(2):
            recv_buf = a2a0_ref if p == 0 else a2a1_ref
            rsem = a2a_recv0 if p == 0 else a2a_recv1
            for s in range(N_DEV):
                wdesc = pltpu.make_async_remote_copy(
                    src_ref=stage_ref.at[0, :, pl.ds(0, 512)],
                    dst_ref=recv_buf.at[s],
                    send_sem=(a2a_send0 if p == 0 else a2a_send1).at[0],
                    recv_sem=rsem.at[s],
                    device_id=(0,),
                    device_id_type=pl.DeviceIdType.MESH,
                )

                @pl.when(my_pos != s)
                def _():
                    wdesc.wait_recv()

        sum0 = a2a0_ref[0].astype(jnp.float32)
        sum1 = a2a1_ref[0].astype(jnp.float32)
        for s in range(1, N_DEV):
            sum0 = sum0 + a2a0_ref[s].astype(jnp.float32)
            sum1 = sum1 + a2a1_ref[s].astype(jnp.float32)

        o0 = my_pos * CHUNK
        o1 = j1 * CHUNK
        out_ref[0, pl.ds(o0, CHUNK), 0:512] = sum0
        out_ref[0, pl.ds(o1, CHUNK), 512:1024] = sum1
        gath_ref[pl.ds(o0, CHUNK), 0:512] = sum0.astype(jnp.bfloat16)
        gath_ref[pl.ds(o1, CHUNK), 512:1024] = sum1.astype(jnp.bfloat16)

        for rdma, is_self in a2a_descs:
            @pl.when(jnp.logical_not(is_self))
            def _():
                rdma.wait_send()

        AG_MASKS = [[1, 3, 4], [4, 1, 3]]
        COLS = [(0, 512), (512, 1024)]
        jown = [my_pos, j1]
        for t in range(3):
            L = CHUNK << t
            rdmas = []
            pbases = []
            for p in range(2):
                m = AG_MASKS[p][t]
                c0, c1 = COLS[p]
                partner = my_pos ^ m
                pb0 = partner & 1
                pb1 = (partner >> 1) & 1
                pb2 = (partner >> 2) & 1
                jp = partner if p == 0 else 4 * pb1 + 2 * pb0 + pb2
                sbase = (jown[p] & ~((1 << t) - 1)) * CHUNK
                pbase = (jp & ~((1 << t) - 1)) * CHUNK
                pbases.append(pbase)
                rdma = pltpu.make_async_remote_copy(
                    src_ref=gath_ref.at[pl.ds(sbase, L), pl.ds(c0, 512)],
                    dst_ref=gath_ref.at[pl.ds(sbase, L), pl.ds(c0, 512)],
                    send_sem=ag_send_sems.at[t, p],
                    recv_sem=ag_recv_sems.at[t, p],
                    device_id=(partner,),
                    device_id_type=pl.DeviceIdType.MESH,
                )
                rdma.start()
                rdmas.append(rdma)
            for p in range(2):
                rdmas[p].wait()
            for p in range(2):
                c0, c1 = COLS[p]
                out_ref[0, pl.ds(pbases[p], L), c0:c1] = (
                    gath_ref[pl.ds(pbases[p], L), c0:c1].astype(jnp.float32))

    return pl.pallas_call(
        body,
        out_shape=jax.ShapeDtypeStruct((1, SQ, D_MODEL), jnp.float32),
        in_specs=[pl.BlockSpec(memory_space=pltpu.VMEM)] * 5,
        out_specs=pl.BlockSpec(memory_space=pltpu.VMEM),
        scratch_shapes=[
            pltpu.VMEM((H_PER, SQ, DH), jnp.bfloat16),
            pltpu.VMEM((H_PER, SKV, DH), jnp.bfloat16),
            pltpu.VMEM((H_PER, SKV, DH), jnp.bfloat16),
            pltpu.VMEM((N_DEV, CHUNK, D_MODEL), jnp.bfloat16),
            pltpu.VMEM((N_DEV, CHUNK, 512), jnp.bfloat16),
            pltpu.VMEM((N_DEV, CHUNK, 512), jnp.bfloat16),
            pltpu.VMEM((SQ, D_MODEL), jnp.bfloat16),
            pltpu.SemaphoreType.DMA((N_DEV,)),
            pltpu.SemaphoreType.DMA((N_DEV,)),
            pltpu.SemaphoreType.DMA((N_DEV,)),
            pltpu.SemaphoreType.DMA((N_DEV,)),
            pltpu.SemaphoreType.DMA((3, 2)),
            pltpu.SemaphoreType.DMA((3, 2)),
        ],
    )(x2, wq_my, k3, v3, wo_my)
